# baseline (speedup 1.0000x reference)
import functools

import jax
import jax.numpy as jnp
from jax import lax
from jax.experimental import pallas as pl
from jax.experimental.pallas import tpu as pltpu
from jax.experimental.pallas import tpu_sc as plsc

N = 10000
DI = 128
DO = 256
E = 320000

NC = 2
NS = 16
NW = NC * NS
K = 128
NB = 80
CH = 40
EPT = NB * K
EPAD = EPT * NW
EPT2 = EPAD // NS
NPAD = 10240
RPT = NPAD // NS
NSLOT = 2


def _degree_body(dst_ref, out_ref, didx_all, deg_local, slab, outbuf, stack_sh):
    c = lax.axis_index("c")
    s = lax.axis_index("s")
    zero16 = jnp.zeros((16,), jnp.float32)
    ones16 = jnp.ones((16,), jnp.float32)

    @pl.when(c == 0)
    def _hist():
        def z(i, carry):
            deg_local[pl.ds(i * 16, 16)] = zero16
            return carry

        lax.fori_loop(0, NPAD // 16, z, 0)
        pltpu.sync_copy(dst_ref.at[pl.ds(s * EPT2, EPT2)], didx_all)

        def h(i, carry):
            iv = didx_all[pl.ds(i * 16, 16)]
            plsc.addupdate_scatter(deg_local, [iv], ones16)
            return carry

        lax.fori_loop(0, EPT2 // 16, h, 0)
        pltpu.sync_copy(deg_local, stack_sh.at[s])

    plsc.subcore_barrier()

    @pl.when(c == 0)
    def _reduce():
        pltpu.sync_copy(stack_sh.at[:, pl.ds(s * RPT, RPT)], slab)

        def r(k, carry):
            acc = slab[0, pl.ds(k * 16, 16)]
            for rr in range(1, NS):
                acc = acc + slab[rr, pl.ds(k * 16, 16)]
            outbuf[pl.ds(k * 16, 16)] = acc
            return carry

        lax.fori_loop(0, RPT // 16, r, 0)
        pltpu.sync_copy(outbuf, out_ref.at[pl.ds(s * RPT, RPT)])


def _scatter_body(src3_ref, dst3_ref, xs_ref, zeros_ref, out_ref,
                  sidx, didx, rows, agg, gsem, ssem):
    c = lax.axis_index("c")
    s = lax.axis_index("s")
    wid = s * NC + c
    pltpu.sync_copy(zeros_ref.at[pl.ds(s * RPT, RPT)],
                    agg.at[pl.ds(s * RPT, RPT)])
    plsc.subcore_barrier()
    base_b = wid * NB

    for ci in range(NB // CH):
        pltpu.sync_copy(src3_ref.at[pl.ds(base_b + ci * CH, CH)], sidx)
        pltpu.sync_copy(dst3_ref.at[pl.ds(base_b + ci * CH, CH)], didx)
        for u in range(NSLOT):
            pltpu.async_copy(xs_ref.at[sidx.at[u, 0]], rows[u], gsem[u])

        def group(g, carry):
            for u in range(NSLOT):
                jj = g * NSLOT + u
                pltpu.make_async_copy(xs_ref.at[sidx.at[u, 0]], rows[u],
                                      gsem[u]).wait()
                pltpu.async_copy(rows[u], agg.at[didx.at[jj, 0]], ssem[u],
                                 add=True)

                @pl.when(jj + NSLOT < CH)
                def _prefetch():
                    pltpu.make_async_copy(rows[u], agg.at[didx.at[u, 0]],
                                          ssem[u]).wait()
                    pltpu.async_copy(xs_ref.at[sidx.at[jj + NSLOT, 0]],
                                     rows[u], gsem[u])

            return carry

        lax.fori_loop(0, CH // NSLOT, group, 0)
        for u in range(NSLOT):
            pltpu.make_async_copy(rows[u], agg.at[didx.at[u, 0]],
                                  ssem[u]).wait()

    plsc.subcore_barrier()
    pltpu.sync_copy(agg.at[pl.ds(s * RPT, RPT)],
                    out_ref.at[c, pl.ds(s * RPT, RPT)])


def _scale_body(x_ref, deg_ref, xs_ref):
    d = deg_ref[...] + 1.0
    xs_ref[...] = x_ref[...] * lax.rsqrt(d)


def _out_body(p_ref, x_ref, deg_ref, w_ref, b_ref, o_ref):
    d = deg_ref[...] + 1.0
    h = (p_ref[0] + p_ref[1]) * lax.rsqrt(d) + x_ref[...] / d
    o_ref[...] = (jnp.dot(h, w_ref[...], preferred_element_type=jnp.float32)
                  + b_ref[...])


_mesh = plsc.VectorSubcoreMesh(core_axis_name="c", subcore_axis_name="s")

_degree_kernel = functools.partial(
    pl.kernel,
    mesh=_mesh,
    compiler_params=pltpu.CompilerParams(needs_layout_passes=False),
    out_type=jax.ShapeDtypeStruct((NPAD,), jnp.float32),
    scratch_types=[
        pltpu.VMEM((EPT2,), jnp.int32),
        pltpu.VMEM((NPAD,), jnp.float32),
        pltpu.VMEM((NS, RPT), jnp.float32),
        pltpu.VMEM((RPT,), jnp.float32),
        pltpu.VMEM_SHARED((NS, NPAD), jnp.float32),
    ],
)(_degree_body)

_scatter_kernel = functools.partial(
    pl.kernel,
    mesh=_mesh,
    out_type=jax.ShapeDtypeStruct((NC, NPAD, DI), jnp.float32),
    scratch_types=[
        pltpu.VMEM((CH, 1, K), jnp.int32),
        pltpu.VMEM((CH, 1, K), jnp.int32),
        [pltpu.VMEM((K, DI), jnp.float32) for _ in range(NSLOT)],
        pltpu.VMEM_SHARED((NPAD, DI), jnp.float32),
        [pltpu.SemaphoreType.DMA for _ in range(NSLOT)],
        [pltpu.SemaphoreType.DMA for _ in range(NSLOT)],
    ],
)(_scatter_body)


def kernel(x, edge_index, W, b):
    src = edge_index[0].astype(jnp.int32)
    dst = edge_index[1].astype(jnp.int32)
    pad = EPAD - E
    src_p = jnp.concatenate([src, jnp.zeros((pad,), jnp.int32)])
    dst_p = jnp.concatenate([dst, jnp.full((pad,), N, jnp.int32)])

    zeros_agg = jnp.zeros((NPAD, DI), jnp.float32)

    degs = _degree_kernel(dst_p)
    deg_col = degs.reshape(NPAD, 1)[:N]

    R = 400
    xs = pl.pallas_call(
        _scale_body,
        grid=(N // R,),
        in_specs=[
            pl.BlockSpec((R, DI), lambda i: (i, 0)),
            pl.BlockSpec((R, 1), lambda i: (i, 0)),
        ],
        out_specs=pl.BlockSpec((R, DI), lambda i: (i, 0)),
        out_shape=jax.ShapeDtypeStruct((N, DI), jnp.float32),
    )(x, deg_col)

    src3 = src_p.reshape(NW * NB, 1, K)
    dst3 = dst_p.reshape(NW * NB, 1, K)
    P = _scatter_kernel(src3, dst3, xs, zeros_agg)
    P_n = P[:, :N]

    out = pl.pallas_call(
        _out_body,
        grid=(N // R,),
        in_specs=[
            pl.BlockSpec((NC, R, DI), lambda i: (0, i, 0)),
            pl.BlockSpec((R, DI), lambda i: (i, 0)),
            pl.BlockSpec((R, 1), lambda i: (i, 0)),
            pl.BlockSpec((DI, DO), lambda i: (0, 0)),
            pl.BlockSpec((1, DO), lambda i: (0, 0)),
        ],
        out_specs=pl.BlockSpec((R, DO), lambda i: (i, 0)),
        out_shape=jax.ShapeDtypeStruct((N, DO), jnp.float32),
    )(P_n, x, deg_col, W, b.reshape(1, DO))
    return out


# speedup vs baseline: 1.0032x; 1.0032x over previous
import functools

import jax
import jax.numpy as jnp
from jax import lax
from jax.experimental import pallas as pl
from jax.experimental.pallas import tpu as pltpu
from jax.experimental.pallas import tpu_sc as plsc

N = 10000
DI = 128
DO = 256
E = 320000

NC = 2
NS = 16
NW = NC * NS
K = 128
NB = 80
CH = 40
EPT = NB * K
EPAD = EPT * NW
EPT2 = EPAD // NS
NPAD = 10240
RPT = NPAD // NS
NSLOT = 2


def _degree_body(dst_ref, out_ref, didx_all, deg_local, slab, outbuf, stack_sh):
    c = lax.axis_index("c")
    s = lax.axis_index("s")
    zero16 = jnp.zeros((16,), jnp.float32)
    ones16 = jnp.ones((16,), jnp.float32)

    @pl.when(c == 0)
    def _hist():
        def z(i, carry):
            deg_local[pl.ds(i * 16, 16)] = zero16
            return carry

        lax.fori_loop(0, NPAD // 16, z, 0)
        pltpu.sync_copy(dst_ref.at[pl.ds(s * EPT2, EPT2)], didx_all)

        def h(i, carry):
            iv = didx_all[pl.ds(i * 16, 16)]
            plsc.addupdate_scatter(deg_local, [iv], ones16)
            return carry

        lax.fori_loop(0, EPT2 // 16, h, 0)
        pltpu.sync_copy(deg_local, stack_sh.at[s])

    plsc.subcore_barrier()

    @pl.when(c == 0)
    def _reduce():
        pltpu.sync_copy(stack_sh.at[:, pl.ds(s * RPT, RPT)], slab)

        def r(k, carry):
            acc = slab[0, pl.ds(k * 16, 16)]
            for rr in range(1, NS):
                acc = acc + slab[rr, pl.ds(k * 16, 16)]
            outbuf[pl.ds(k * 16, 16)] = acc
            return carry

        lax.fori_loop(0, RPT // 16, r, 0)
        pltpu.sync_copy(outbuf, out_ref.at[pl.ds(s * RPT, RPT)])


def _scatter_body(src_ref, dst_ref, xs_ref, zeros_ref, out_ref,
                  sidx, didx, rows, agg, gsem, ssem, isem):
    c = lax.axis_index("c")
    s = lax.axis_index("s")
    wid = s * NC + c
    pltpu.sync_copy(zeros_ref.at[pl.ds(s * RPT, RPT)],
                    agg.at[pl.ds(s * RPT, RPT)])
    plsc.subcore_barrier()
    base = wid * EPT

    def idx_refs(j):
        off = pl.multiple_of(base + j * K, 8)
        return src_ref.at[pl.ds(off, K)], dst_ref.at[pl.ds(off, K)]

    for u in range(2):
        sr, dr = idx_refs(u)
        pltpu.sync_copy(sr, sidx[u])
        pltpu.sync_copy(dr, didx[u])
        pltpu.async_copy(xs_ref.at[sidx[u]], rows[u], gsem[u])
    for v in range(2, 4):
        sr, dr = idx_refs(v)
        pltpu.async_copy(sr, sidx[v], isem[v])
        pltpu.async_copy(dr, didx[v], isem[v])

    def group(g, carry):
        for u4 in range(4):
            j = g * 4 + u4
            u = u4 % NSLOT
            v = u4
            v2 = (u4 + 2) % 4
            pltpu.make_async_copy(xs_ref.at[sidx[v]], rows[u],
                                  gsem[u]).wait()
            pltpu.async_copy(rows[u], agg.at[didx[v]], ssem[u], add=True)

            @pl.when(j + 2 < NB)
            def _next_gather():
                pltpu.make_async_copy(rows[u], agg.at[didx[v]],
                                      ssem[u]).wait()
                sr2, dr2 = idx_refs(j + 2)
                pltpu.make_async_copy(sr2, sidx[v2], isem[v2]).wait()
                pltpu.make_async_copy(dr2, didx[v2], isem[v2]).wait()
                pltpu.async_copy(xs_ref.at[sidx[v2]], rows[u], gsem[u])

                @pl.when(j + 4 < NB)
                def _prefetch_idx():
                    sr4, dr4 = idx_refs(j + 4)
                    pltpu.async_copy(sr4, sidx[v], isem[v])
                    pltpu.async_copy(dr4, didx[v], isem[v])

        return carry

    lax.fori_loop(0, NB // 4, group, 0)
    for u in range(NSLOT):
        pltpu.make_async_copy(rows[u], agg.at[didx[u]], ssem[u]).wait()

    plsc.subcore_barrier()
    pltpu.sync_copy(agg.at[pl.ds(s * RPT, RPT)],
                    out_ref.at[c, pl.ds(s * RPT, RPT)])


def _scale_body(x_ref, deg_ref, xs_ref):
    d = deg_ref[...] + 1.0
    xs_ref[...] = x_ref[...] * lax.rsqrt(d)


def _out_body(p_ref, x_ref, deg_ref, w_ref, b_ref, o_ref):
    d = deg_ref[...] + 1.0
    h = (p_ref[0] + p_ref[1]) * lax.rsqrt(d) + x_ref[...] / d
    o_ref[...] = (jnp.dot(h, w_ref[...], preferred_element_type=jnp.float32)
                  + b_ref[...])


_mesh = plsc.VectorSubcoreMesh(core_axis_name="c", subcore_axis_name="s")

_degree_kernel = functools.partial(
    pl.kernel,
    mesh=_mesh,
    compiler_params=pltpu.CompilerParams(needs_layout_passes=False),
    out_type=jax.ShapeDtypeStruct((NPAD,), jnp.float32),
    scratch_types=[
        pltpu.VMEM((EPT2,), jnp.int32),
        pltpu.VMEM((NPAD,), jnp.float32),
        pltpu.VMEM((NS, RPT), jnp.float32),
        pltpu.VMEM((RPT,), jnp.float32),
        pltpu.VMEM_SHARED((NS, NPAD), jnp.float32),
    ],
)(_degree_body)

_scatter_kernel = functools.partial(
    pl.kernel,
    mesh=_mesh,
    out_type=jax.ShapeDtypeStruct((NC, NPAD, DI), jnp.float32),
    scratch_types=[
        [pltpu.VMEM((K,), jnp.int32) for _ in range(4)],
        [pltpu.VMEM((K,), jnp.int32) for _ in range(4)],
        [pltpu.VMEM((K, DI), jnp.float32) for _ in range(NSLOT)],
        pltpu.VMEM_SHARED((NPAD, DI), jnp.float32),
        [pltpu.SemaphoreType.DMA for _ in range(NSLOT)],
        [pltpu.SemaphoreType.DMA for _ in range(NSLOT)],
        [pltpu.SemaphoreType.DMA for _ in range(4)],
    ],
)(_scatter_body)


def kernel(x, edge_index, W, b):
    src = edge_index[0].astype(jnp.int32)
    dst = edge_index[1].astype(jnp.int32)
    pad = EPAD - E
    src_p = jnp.concatenate([src, jnp.zeros((pad,), jnp.int32)])
    dst_p = jnp.concatenate([dst, jnp.full((pad,), N, jnp.int32)])

    zeros_agg = jnp.zeros((NPAD, DI), jnp.float32)

    degs = _degree_kernel(dst_p)
    deg_col = degs.reshape(NPAD, 1)[:N]

    R = 400
    xs = pl.pallas_call(
        _scale_body,
        grid=(N // R,),
        in_specs=[
            pl.BlockSpec((R, DI), lambda i: (i, 0)),
            pl.BlockSpec((R, 1), lambda i: (i, 0)),
        ],
        out_specs=pl.BlockSpec((R, DI), lambda i: (i, 0)),
        out_shape=jax.ShapeDtypeStruct((N, DI), jnp.float32),
    )(x, deg_col)

    P = _scatter_kernel(src_p, dst_p, xs, zeros_agg)
    P_n = P[:, :N]

    out = pl.pallas_call(
        _out_body,
        grid=(N // R,),
        in_specs=[
            pl.BlockSpec((NC, R, DI), lambda i: (0, i, 0)),
            pl.BlockSpec((R, DI), lambda i: (i, 0)),
            pl.BlockSpec((R, 1), lambda i: (i, 0)),
            pl.BlockSpec((DI, DO), lambda i: (0, 0)),
            pl.BlockSpec((1, DO), lambda i: (0, 0)),
        ],
        out_specs=pl.BlockSpec((R, DO), lambda i: (i, 0)),
        out_shape=jax.ShapeDtypeStruct((N, DO), jnp.float32),
    )(P_n, x, deg_col, W, b.reshape(1, DO))
    return out


# speedup vs baseline: 1.0108x; 1.0076x over previous
import functools

import jax
import jax.numpy as jnp
from jax import lax
from jax.experimental import pallas as pl
from jax.experimental.pallas import tpu as pltpu
from jax.experimental.pallas import tpu_sc as plsc

N = 10000
DI = 128
DO = 256
E = 320000

NC = 2
NS = 16
NW = NC * NS
K = 64
NB = 160
EPT = NB * K
EPAD = EPT * NW
EPT2 = EPAD // NS
NPAD = 10240
RPT = NPAD // NS
NSLOT = 4


def _degree_body(dst_ref, out_ref, didx_all, deg_local, slab, outbuf, stack_sh):
    c = lax.axis_index("c")
    s = lax.axis_index("s")
    zero16 = jnp.zeros((16,), jnp.float32)
    ones16 = jnp.ones((16,), jnp.float32)

    @pl.when(c == 0)
    def _hist():
        def z(i, carry):
            deg_local[pl.ds(i * 16, 16)] = zero16
            return carry

        lax.fori_loop(0, NPAD // 16, z, 0)
        pltpu.sync_copy(dst_ref.at[pl.ds(s * EPT2, EPT2)], didx_all)

        def h(i, carry):
            iv = didx_all[pl.ds(i * 16, 16)]
            plsc.addupdate_scatter(deg_local, [iv], ones16)
            return carry

        lax.fori_loop(0, EPT2 // 16, h, 0)
        pltpu.sync_copy(deg_local, stack_sh.at[s])

    plsc.subcore_barrier()

    @pl.when(c == 0)
    def _reduce():
        pltpu.sync_copy(stack_sh.at[:, pl.ds(s * RPT, RPT)], slab)

        def r(k, carry):
            acc = slab[0, pl.ds(k * 16, 16)]
            for rr in range(1, NS):
                acc = acc + slab[rr, pl.ds(k * 16, 16)]
            outbuf[pl.ds(k * 16, 16)] = acc
            return carry

        lax.fori_loop(0, RPT // 16, r, 0)
        pltpu.sync_copy(outbuf, out_ref.at[pl.ds(s * RPT, RPT)])


def _scatter_body(src_ref, dst_ref, xs_ref, zeros_ref, out_ref,
                  sidx, didx, rows, agg, gsem, ssem):
    c = lax.axis_index("c")
    s = lax.axis_index("s")
    wid = s * NC + c
    pltpu.sync_copy(zeros_ref.at[pl.ds(s * RPT, RPT)],
                    agg.at[pl.ds(s * RPT, RPT)])
    plsc.subcore_barrier()
    base = wid * EPT

    def load_and_gather(j, slot):
        off = pl.multiple_of(base + j * K, 8)
        pltpu.sync_copy(src_ref.at[pl.ds(off, K)], sidx[slot])
        pltpu.sync_copy(dst_ref.at[pl.ds(off, K)], didx[slot])
        pltpu.async_copy(xs_ref.at[sidx[slot]], rows[slot], gsem[slot])

    for u in range(2):
        load_and_gather(u, u)

    def group(g, carry):
        for u4 in range(NSLOT):
            j = g * NSLOT + u4
            u2 = (u4 + 2) % NSLOT
            pltpu.make_async_copy(xs_ref.at[sidx[u4]], rows[u4],
                                  gsem[u4]).wait()
            pltpu.async_copy(rows[u4], agg.at[didx[u4]], ssem[u4], add=True)

            @pl.when(j >= 2)
            def _wait_old_scatter():
                pltpu.make_async_copy(rows[u2], agg.at[didx[u2]],
                                      ssem[u2]).wait()

            @pl.when(j + 2 < NB)
            def _next_gather():
                load_and_gather(j + 2, u2)

        return carry

    lax.fori_loop(0, NB // NSLOT, group, 0)
    for u in (2, 3):
        pltpu.make_async_copy(rows[u], agg.at[didx[u]], ssem[u]).wait()

    plsc.subcore_barrier()
    pltpu.sync_copy(agg.at[pl.ds(s * RPT, RPT)],
                    out_ref.at[c, pl.ds(s * RPT, RPT)])


def _scale_body(x_ref, deg_ref, xs_ref):
    d = deg_ref[...] + 1.0
    xs_ref[...] = x_ref[...] * lax.rsqrt(d)


def _out_body(p_ref, x_ref, deg_ref, w_ref, b_ref, o_ref):
    d = deg_ref[...] + 1.0
    h = (p_ref[0] + p_ref[1]) * lax.rsqrt(d) + x_ref[...] / d
    o_ref[...] = (jnp.dot(h, w_ref[...], preferred_element_type=jnp.float32)
                  + b_ref[...])


_mesh = plsc.VectorSubcoreMesh(core_axis_name="c", subcore_axis_name="s")

_degree_kernel = functools.partial(
    pl.kernel,
    mesh=_mesh,
    compiler_params=pltpu.CompilerParams(needs_layout_passes=False),
    out_type=jax.ShapeDtypeStruct((NPAD,), jnp.float32),
    scratch_types=[
        pltpu.VMEM((EPT2,), jnp.int32),
        pltpu.VMEM((NPAD,), jnp.float32),
        pltpu.VMEM((NS, RPT), jnp.float32),
        pltpu.VMEM((RPT,), jnp.float32),
        pltpu.VMEM_SHARED((NS, NPAD), jnp.float32),
    ],
)(_degree_body)

_scatter_kernel = functools.partial(
    pl.kernel,
    mesh=_mesh,
    out_type=jax.ShapeDtypeStruct((NC, NPAD, DI), jnp.float32),
    scratch_types=[
        [pltpu.VMEM((K,), jnp.int32) for _ in range(4)],
        [pltpu.VMEM((K,), jnp.int32) for _ in range(4)],
        [pltpu.VMEM((K, DI), jnp.float32) for _ in range(NSLOT)],
        pltpu.VMEM_SHARED((NPAD, DI), jnp.float32),
        [pltpu.SemaphoreType.DMA for _ in range(NSLOT)],
        [pltpu.SemaphoreType.DMA for _ in range(NSLOT)],
    ],
)(_scatter_body)


def kernel(x, edge_index, W, b):
    src = edge_index[0].astype(jnp.int32)
    dst = edge_index[1].astype(jnp.int32)
    pad = EPAD - E
    src_p = jnp.concatenate([src, jnp.zeros((pad,), jnp.int32)])
    dst_p = jnp.concatenate([dst, jnp.full((pad,), N, jnp.int32)])

    zeros_agg = jnp.zeros((NPAD, DI), jnp.float32)

    degs = _degree_kernel(dst_p)
    deg_col = degs.reshape(NPAD, 1)[:N]

    R = 400
    xs = pl.pallas_call(
        _scale_body,
        grid=(N // R,),
        in_specs=[
            pl.BlockSpec((R, DI), lambda i: (i, 0)),
            pl.BlockSpec((R, 1), lambda i: (i, 0)),
        ],
        out_specs=pl.BlockSpec((R, DI), lambda i: (i, 0)),
        out_shape=jax.ShapeDtypeStruct((N, DI), jnp.float32),
    )(x, deg_col)

    P = _scatter_kernel(src_p, dst_p, xs, zeros_agg)
    P_n = P[:, :N]

    out = pl.pallas_call(
        _out_body,
        grid=(N // R,),
        in_specs=[
            pl.BlockSpec((NC, R, DI), lambda i: (0, i, 0)),
            pl.BlockSpec((R, DI), lambda i: (i, 0)),
            pl.BlockSpec((R, 1), lambda i: (i, 0)),
            pl.BlockSpec((DI, DO), lambda i: (0, 0)),
            pl.BlockSpec((1, DO), lambda i: (0, 0)),
        ],
        out_specs=pl.BlockSpec((R, DO), lambda i: (i, 0)),
        out_shape=jax.ShapeDtypeStruct((N, DO), jnp.float32),
    )(P_n, x, deg_col, W, b.reshape(1, DO))
    return out


# speedup vs baseline: 1.0223x; 1.0114x over previous
import functools

import jax
import jax.numpy as jnp
from jax import lax
from jax.experimental import pallas as pl
from jax.experimental.pallas import tpu as pltpu
from jax.experimental.pallas import tpu_sc as plsc

N = 10000
DI = 128
DO = 256
E = 320000

NC = 2
NS = 16
NW = NC * NS
K = 64
NB = 160
EPT = NB * K
EPAD = EPT * NW
EPT2 = EPAD // NS
NPAD = 10240
RPT = NPAD // NS
NSLOT = 4


def _degree_body(dst_ref, out_ref, didx_all, deg_local, slab, outbuf, stack_sh):
    c = lax.axis_index("c")
    s = lax.axis_index("s")
    zero16 = jnp.zeros((16,), jnp.float32)
    ones16 = jnp.ones((16,), jnp.float32)

    @pl.when(c == 0)
    def _hist():
        def z(i, carry):
            deg_local[pl.ds(i * 16, 16)] = zero16
            return carry

        lax.fori_loop(0, NPAD // 16, z, 0)
        pltpu.sync_copy(dst_ref.at[pl.ds(s * EPT2, EPT2)], didx_all)

        def h(i, carry):
            iv = didx_all[pl.ds(i * 16, 16)]
            plsc.addupdate_scatter(deg_local, [iv], ones16)
            return carry

        lax.fori_loop(0, EPT2 // 16, h, 0)
        pltpu.sync_copy(deg_local, stack_sh.at[s])

    plsc.subcore_barrier()

    @pl.when(c == 0)
    def _reduce():
        pltpu.sync_copy(stack_sh.at[:, pl.ds(s * RPT, RPT)], slab)

        def r(k, carry):
            acc = slab[0, pl.ds(k * 16, 16)]
            for rr in range(1, NS):
                acc = acc + slab[rr, pl.ds(k * 16, 16)]
            outbuf[pl.ds(k * 16, 16)] = acc
            return carry

        lax.fori_loop(0, RPT // 16, r, 0)
        pltpu.sync_copy(outbuf, out_ref.at[pl.ds(s * RPT, RPT)])


def _scatter_body(src_ref, dst_ref, xs_ref, zeros_ref, out_ref,
                  sidx, didx, rows, agg, gsem, ssem):
    c = lax.axis_index("c")
    s = lax.axis_index("s")
    wid = s * NC + c
    pltpu.sync_copy(zeros_ref.at[pl.ds(s * RPT, RPT)],
                    agg.at[pl.ds(s * RPT, RPT)])
    plsc.subcore_barrier()
    base = wid * EPT

    def load_and_gather(j, slot):
        off = pl.multiple_of(base + j * K, 8)
        pltpu.sync_copy(src_ref.at[pl.ds(off, K)], sidx[slot])
        pltpu.sync_copy(dst_ref.at[pl.ds(off, K)], didx[slot])
        pltpu.async_copy(xs_ref.at[sidx[slot]], rows[slot], gsem[slot])

    for u in range(2):
        load_and_gather(u, u)

    def group(g, carry):
        for u4 in range(NSLOT):
            j = g * NSLOT + u4
            u2 = (u4 + 2) % NSLOT
            pltpu.make_async_copy(xs_ref.at[sidx[u4]], rows[u4],
                                  gsem[u4]).wait()
            pltpu.async_copy(rows[u4], agg.at[didx[u4]], ssem[u4], add=True)

            @pl.when(j >= 2)
            def _wait_old_scatter():
                pltpu.make_async_copy(rows[u2], agg.at[didx[u2]],
                                      ssem[u2]).wait()

            @pl.when(j + 2 < NB)
            def _next_gather():
                load_and_gather(j + 2, u2)

        return carry

    lax.fori_loop(0, NB // NSLOT, group, 0)
    for u in (2, 3):
        pltpu.make_async_copy(rows[u], agg.at[didx[u]], ssem[u]).wait()

    plsc.subcore_barrier()
    pltpu.sync_copy(agg.at[pl.ds(s * RPT, RPT)],
                    out_ref.at[c, pl.ds(s * RPT, RPT)])


def _scale_body(x_ref, deg_ref, xs_ref):
    d = deg_ref[...] + 1.0
    xs_ref[...] = x_ref[...] * lax.rsqrt(d)


def _out_body(p_ref, x_ref, deg_ref, w_ref, b_ref, o_ref):
    d = deg_ref[...] + 1.0
    h = (p_ref[0] + p_ref[1]) * lax.rsqrt(d) + x_ref[...] / d
    o_ref[...] = (jnp.dot(h, w_ref[...], preferred_element_type=jnp.float32)
                  + b_ref[...])


_mesh = plsc.VectorSubcoreMesh(core_axis_name="c", subcore_axis_name="s")

_degree_kernel = functools.partial(
    pl.kernel,
    mesh=_mesh,
    compiler_params=pltpu.CompilerParams(needs_layout_passes=False),
    out_type=jax.ShapeDtypeStruct((NPAD,), jnp.float32),
    scratch_types=[
        pltpu.VMEM((EPT2,), jnp.int32),
        pltpu.VMEM((NPAD,), jnp.float32),
        pltpu.VMEM((NS, RPT), jnp.float32),
        pltpu.VMEM((RPT,), jnp.float32),
        pltpu.VMEM_SHARED((NS, NPAD), jnp.float32),
    ],
)(_degree_body)

_scatter_kernel = functools.partial(
    pl.kernel,
    mesh=_mesh,
    out_type=jax.ShapeDtypeStruct((NC, NPAD, DI), jnp.float32),
    scratch_types=[
        [pltpu.VMEM((K,), jnp.int32) for _ in range(4)],
        [pltpu.VMEM((K,), jnp.int32) for _ in range(4)],
        [pltpu.VMEM((K, DI), jnp.float32) for _ in range(NSLOT)],
        pltpu.VMEM_SHARED((NPAD, DI), jnp.float32),
        [pltpu.SemaphoreType.DMA for _ in range(NSLOT)],
        [pltpu.SemaphoreType.DMA for _ in range(NSLOT)],
    ],
)(_scatter_body)


def kernel(x, edge_index, W, b):
    src = edge_index[0].astype(jnp.int32)
    dst = edge_index[1].astype(jnp.int32)
    pad = EPAD - E
    src_p = jnp.concatenate([src, jnp.zeros((pad,), jnp.int32)])
    trash = N + (jnp.arange(pad, dtype=jnp.int32) % (NPAD - N))
    dst_p = jnp.concatenate([dst, trash])

    zeros_agg = jnp.zeros((NPAD, DI), jnp.float32)

    degs = _degree_kernel(dst_p)
    deg_col = degs.reshape(NPAD, 1)[:N]

    R = 400
    xs = pl.pallas_call(
        _scale_body,
        grid=(N // R,),
        in_specs=[
            pl.BlockSpec((R, DI), lambda i: (i, 0)),
            pl.BlockSpec((R, 1), lambda i: (i, 0)),
        ],
        out_specs=pl.BlockSpec((R, DI), lambda i: (i, 0)),
        out_shape=jax.ShapeDtypeStruct((N, DI), jnp.float32),
    )(x, deg_col)

    P = _scatter_kernel(src_p, dst_p, xs, zeros_agg)
    P_n = P[:, :N]

    out = pl.pallas_call(
        _out_body,
        grid=(N // R,),
        in_specs=[
            pl.BlockSpec((NC, R, DI), lambda i: (0, i, 0)),
            pl.BlockSpec((R, DI), lambda i: (i, 0)),
            pl.BlockSpec((R, 1), lambda i: (i, 0)),
            pl.BlockSpec((DI, DO), lambda i: (0, 0)),
            pl.BlockSpec((1, DO), lambda i: (0, 0)),
        ],
        out_specs=pl.BlockSpec((R, DO), lambda i: (i, 0)),
        out_shape=jax.ShapeDtypeStruct((N, DO), jnp.float32),
    )(P_n, x, deg_col, W, b.reshape(1, DO))
    return out


# speedup vs baseline: 1.5998x; 1.5650x over previous
import functools

import jax
import jax.numpy as jnp
from jax import lax
from jax.experimental import pallas as pl
from jax.experimental.pallas import tpu as pltpu
from jax.experimental.pallas import tpu_sc as plsc

N = 10000
DI = 128
DO = 256
E = 320000

NC = 2
NS = 16
NW = NC * NS
K = 128
NB = 79
EPT = NB * K
EPAD = EPT * NW
EPT2 = EPAD // NS
NPAD = 10240
RPT = NPAD // NS
NSLOT = 2


def _degree_body(dst_ref, out_ref, didx_all, deg_local, slab, outbuf, stack_sh):
    c = lax.axis_index("c")
    s = lax.axis_index("s")
    zero16 = jnp.zeros((16,), jnp.float32)
    ones16 = jnp.ones((16,), jnp.float32)

    @pl.when(c == 0)
    def _hist():
        def z(i, carry):
            deg_local[pl.ds(i * 16, 16)] = zero16
            return carry

        lax.fori_loop(0, NPAD // 16, z, 0)
        pltpu.sync_copy(dst_ref.at[pl.ds(s * EPT2, EPT2)], didx_all)

        def h(i, carry):
            iv = didx_all[pl.ds(i * 16, 16)]
            plsc.addupdate_scatter(deg_local, [iv], ones16)
            return carry

        lax.fori_loop(0, EPT2 // 16, h, 0)
        pltpu.sync_copy(deg_local, stack_sh.at[s])

    plsc.subcore_barrier()

    @pl.when(c == 0)
    def _reduce():
        pltpu.sync_copy(stack_sh.at[:, pl.ds(s * RPT, RPT)], slab)

        def r(k, carry):
            acc = slab[0, pl.ds(k * 16, 16)]
            for rr in range(1, NS):
                acc = acc + slab[rr, pl.ds(k * 16, 16)]
            outbuf[pl.ds(k * 16, 16)] = acc
            return carry

        lax.fori_loop(0, RPT // 16, r, 0)
        pltpu.sync_copy(outbuf, out_ref.at[pl.ds(s * RPT, RPT)])


def _scatter_body(src_ref, dst_ref, xs_ref, zeros_ref, out_ref,
                  sidx, didx, rows, agg, gsem, ssem):
    c = lax.axis_index("c")
    s = lax.axis_index("s")
    wid = s * NC + c
    pltpu.sync_copy(zeros_ref.at[pl.ds(s * RPT, RPT)],
                    agg.at[pl.ds(s * RPT, RPT)])
    plsc.subcore_barrier()
    base = wid * EPT

    def load_and_gather(j, slot):
        off = pl.multiple_of(base + j * K, 8)
        pltpu.sync_copy(src_ref.at[pl.ds(off, K)], sidx[slot])
        pltpu.sync_copy(dst_ref.at[pl.ds(off, K)], didx[slot])
        pltpu.async_copy(xs_ref.at[sidx[slot]], rows[slot], gsem[slot])

    for u in range(NSLOT):
        load_and_gather(u, u)

    def group(g, carry):
        for u in range(NSLOT):
            j = g * NSLOT + u
            pltpu.make_async_copy(xs_ref.at[sidx[u]], rows[u],
                                  gsem[u]).wait()
            pltpu.async_copy(rows[u], agg.at[didx[u]], ssem[u], add=True)

            @pl.when(j + NSLOT < NB - 1)
            def _prefetch():
                pltpu.make_async_copy(rows[u], agg.at[didx[u]],
                                      ssem[u]).wait()
                load_and_gather(j + NSLOT, u)

        return carry

    lax.fori_loop(0, (NB - 1) // NSLOT, group, 0)
    pltpu.make_async_copy(rows[0], agg.at[didx[0]], ssem[0]).wait()
    load_and_gather(NB - 1, 0)
    pltpu.make_async_copy(xs_ref.at[sidx[0]], rows[0], gsem[0]).wait()
    pltpu.async_copy(rows[0], agg.at[didx[0]], ssem[0], add=True)
    for u in range(NSLOT):
        pltpu.make_async_copy(rows[u], agg.at[didx[u]], ssem[u]).wait()

    plsc.subcore_barrier()
    pltpu.sync_copy(agg.at[pl.ds(s * RPT, RPT)],
                    out_ref.at[c, pl.ds(s * RPT, RPT)])


def _scale_body(x_ref, deg_ref, xs_ref):
    d = deg_ref[...] + 1.0
    xs_ref[...] = x_ref[...] * lax.rsqrt(d)


def _out_body(p_ref, x_ref, deg_ref, w_ref, b_ref, o_ref):
    d = deg_ref[...] + 1.0
    h = (p_ref[0] + p_ref[1]) * lax.rsqrt(d) + x_ref[...] / d
    o_ref[...] = (jnp.dot(h, w_ref[...], preferred_element_type=jnp.float32)
                  + b_ref[...])


_mesh = plsc.VectorSubcoreMesh(core_axis_name="c", subcore_axis_name="s")

_degree_kernel = functools.partial(
    pl.kernel,
    mesh=_mesh,
    compiler_params=pltpu.CompilerParams(needs_layout_passes=False),
    out_type=jax.ShapeDtypeStruct((NPAD,), jnp.float32),
    scratch_types=[
        pltpu.VMEM((EPT2,), jnp.int32),
        pltpu.VMEM((NPAD,), jnp.float32),
        pltpu.VMEM((NS, RPT), jnp.float32),
        pltpu.VMEM((RPT,), jnp.float32),
        pltpu.VMEM_SHARED((NS, NPAD), jnp.float32),
    ],
)(_degree_body)

_scatter_kernel = functools.partial(
    pl.kernel,
    mesh=_mesh,
    out_type=jax.ShapeDtypeStruct((NC, NPAD, DI), jnp.float32),
    scratch_types=[
        [pltpu.VMEM((K,), jnp.int32) for _ in range(4)],
        [pltpu.VMEM((K,), jnp.int32) for _ in range(4)],
        [pltpu.VMEM((K, DI), jnp.float32) for _ in range(NSLOT)],
        pltpu.VMEM_SHARED((NPAD, DI), jnp.float32),
        [pltpu.SemaphoreType.DMA for _ in range(NSLOT)],
        [pltpu.SemaphoreType.DMA for _ in range(NSLOT)],
    ],
)(_scatter_body)


def kernel(x, edge_index, W, b):
    src = edge_index[0].astype(jnp.int32)
    dst = edge_index[1].astype(jnp.int32)
    pad = EPAD - E
    src_p = jnp.concatenate([src, jnp.zeros((pad,), jnp.int32)])
    trash = N + (jnp.arange(pad, dtype=jnp.int32) % (NPAD - N))
    dst_p = jnp.concatenate([dst, trash])

    zeros_agg = jnp.zeros((NPAD, DI), jnp.float32)

    degs = _degree_kernel(dst_p)
    deg_col = degs.reshape(NPAD, 1)[:N]

    R = 400
    xs = pl.pallas_call(
        _scale_body,
        grid=(N // R,),
        in_specs=[
            pl.BlockSpec((R, DI), lambda i: (i, 0)),
            pl.BlockSpec((R, 1), lambda i: (i, 0)),
        ],
        out_specs=pl.BlockSpec((R, DI), lambda i: (i, 0)),
        out_shape=jax.ShapeDtypeStruct((N, DI), jnp.float32),
    )(x, deg_col)

    P = _scatter_kernel(src_p, dst_p, xs, zeros_agg)
    P_n = P[:, :N]

    out = pl.pallas_call(
        _out_body,
        grid=(N // R,),
        in_specs=[
            pl.BlockSpec((NC, R, DI), lambda i: (0, i, 0)),
            pl.BlockSpec((R, DI), lambda i: (i, 0)),
            pl.BlockSpec((R, 1), lambda i: (i, 0)),
            pl.BlockSpec((DI, DO), lambda i: (0, 0)),
            pl.BlockSpec((1, DO), lambda i: (0, 0)),
        ],
        out_specs=pl.BlockSpec((R, DO), lambda i: (i, 0)),
        out_shape=jax.ShapeDtypeStruct((N, DO), jnp.float32),
    )(P_n, x, deg_col, W, b.reshape(1, DO))
    return out
